# baseline (speedup 1.0000x reference)
import functools

import jax
import jax.numpy as jnp
from jax import lax
from jax.experimental import pallas as pl
from jax.experimental.pallas import tpu as pltpu
from jax.experimental.pallas import tpu_sc as plsc

_B, _L, _E = 4096, 50, 64
_N = _B * _L
_V = 100000
_NC, _NS = 2, 16
_NW = _NC * _NS
_BPW = _B // _NW
_NB = 4
_NBLK = _BPW // _NB
_ROWS = _NB * _L
_PADR = 2000


def _pad_body(x_ref, o_ref):
  o_ref[:, : _E] = x_ref[...]
  o_ref[:, _E:] = jnp.zeros((_PADR, 128 - _E), jnp.float32)


_pad_call = pl.pallas_call(
    _pad_body,
    out_shape=jax.ShapeDtypeStruct((_V, 128), jnp.float32),
    grid=(_V // _PADR,),
    in_specs=[pl.BlockSpec((_PADR, _E), lambda i: (i, 0))],
    out_specs=pl.BlockSpec((_PADR, 128), lambda i: (i, 0)),
)


def _make_kernel():
  mesh = plsc.VectorSubcoreMesh(
      core_axis_name="c", subcore_axis_name="s",
      num_cores=_NC, num_subcores=_NS)

  @functools.partial(
      pl.kernel,
      out_type=jax.ShapeDtypeStruct((_B, _L, _E), jnp.float32),
      mesh=mesh,
      compiler_params=pltpu.CompilerParams(use_tc_tiling_on_sc=True),
      scratch_types=[
          pltpu.VMEM((_BPW * _L,), jnp.int32),
          pltpu.VMEM((_ROWS, 128), jnp.float32),
          pltpu.VMEM((_ROWS, 128), jnp.float32),
          pltpu.VMEM((_NB, _L, _E), jnp.float32),
          pltpu.VMEM((_NB, _L, _E), jnp.float32),
          pltpu.SemaphoreType.DMA,
          pltpu.SemaphoreType.DMA,
          pltpu.SemaphoreType.DMA,
          pltpu.SemaphoreType.DMA,
      ],
  )
  def emb_kernel(idx_hbm, table_hbm, out_hbm, idx_all,
                 gbuf0, gbuf1, obuf0, obuf1, g0, g1, o0, o1):
    wid = lax.axis_index("s") * _NC + lax.axis_index("c")
    b0 = wid * _BPW
    gbufs = (gbuf0, gbuf1)
    obufs = (obuf0, obuf1)
    gsems = (g0, g1)
    osems = (o0, o1)

    pltpu.sync_copy(idx_hbm.at[pl.ds(b0 * _L, _BPW * _L)], idx_all)

    def gather(k, s):
      pltpu.async_copy(
          table_hbm.at[idx_all.at[pl.ds(k * _ROWS, _ROWS)]],
          gbufs[s], gsems[s])

    def wait_gather(s):
      pltpu.make_async_copy(
          table_hbm.at[pl.ds(0, _ROWS)], gbufs[s], gsems[s]).wait()

    def repack(s):
      gb, ob = gbufs[s], obufs[s]
      for j in range(_ROWS):
        for h in range(_E // 16):
          ob[j // _L, j % _L, pl.ds(h * 16, 16)] = gb[j, pl.ds(h * 16, 16)]

    def store(k, s):
      pltpu.async_copy(obufs[s], out_hbm.at[pl.ds(b0 + k * _NB, _NB)],
                       osems[s])

    def wait_store(s):
      pltpu.make_async_copy(obufs[s], out_hbm.at[pl.ds(b0, _NB)],
                            osems[s]).wait()

    gather(0, 0)
    gather(1, 1)

    def half(i, k, s):
      wait_gather(s)
      pl.when(i > 0)(lambda: wait_store(s))
      repack(s)
      store(k, s)
      pl.when(i < _NBLK // 2 - 1)(lambda: gather(k + 2, s))

    def body(i, _):
      half(i, 2 * i, 0)
      half(i, 2 * i + 1, 1)
      return 0

    lax.fori_loop(0, _NBLK // 2, body, 0)

    wait_store(0)
    wait_store(1)

  return emb_kernel


_EMB = _make_kernel()


@jax.jit
def kernel(src_idx, tar_idx, src_table, tar_table):
  sp = _pad_call(src_table)
  tp = _pad_call(tar_table)
  src_out = _EMB(src_idx.reshape(_N), sp)
  tar_out = _EMB(tar_idx.reshape(_N), tp)
  return (src_out, tar_out)


# speedup vs baseline: 1.0449x; 1.0449x over previous
import functools

import jax
import jax.numpy as jnp
from jax import lax
from jax.experimental import pallas as pl
from jax.experimental.pallas import tpu as pltpu
from jax.experimental.pallas import tpu_sc as plsc

_B, _L, _E = 4096, 50, 64
_V = 100000
_NC, _NS = 2, 16
_NW = _NC * _NS
_BPW = _B // _NW
_NB = 4
_NBLK = _BPW // _NB
_GROW = 56


def _make_kernel():
  mesh = plsc.VectorSubcoreMesh(
      core_axis_name="c", subcore_axis_name="s",
      num_cores=_NC, num_subcores=_NS)

  @functools.partial(
      pl.kernel,
      out_type=(
          jax.ShapeDtypeStruct((_B, _L, _E), jnp.float32),
          jax.ShapeDtypeStruct((_B, _L, _E), jnp.float32),
      ),
      mesh=mesh,
      compiler_params=pltpu.CompilerParams(use_tc_tiling_on_sc=True),
      scratch_types=[
          pltpu.VMEM((_BPW, _L), jnp.int32),
          pltpu.VMEM((_NB * _GROW, 128), jnp.float32),
          pltpu.VMEM((_NB * _GROW, 128), jnp.float32),
          pltpu.VMEM((_NB, _L, _E), jnp.float32),
          pltpu.SemaphoreType.DMA,
          pltpu.SemaphoreType.DMA,
      ],
  )
  def emb_kernel(src_idx, tar_idx, src_table, tar_table,
                 src_out, tar_out, idx_all, gbuf0, gbuf1, obuf, g0, g1):
    wid = lax.axis_index("s") * _NC + lax.axis_index("c")
    b0 = wid * _BPW
    gbufs = (gbuf0, gbuf1)
    gsems = (g0, g1)

    def run_table(idx_hbm, table_hbm, out_hbm):
      pltpu.sync_copy(idx_hbm.at[pl.ds(b0, _BPW), :], idx_all)

      def gather(k, s):
        for j2 in range(_NB):
          pltpu.async_copy(
              table_hbm.at[idx_all.at[k * _NB + j2, pl.ds(0, _L)]],
              gbufs[s].at[pl.ds(j2 * _GROW, _L)], gsems[s])

      def wait_gather(s):
        for j2 in range(_NB):
          pltpu.make_async_copy(
              table_hbm.at[idx_all.at[j2, pl.ds(0, _L)]],
              gbufs[s].at[pl.ds(j2 * _GROW, _L)], gsems[s]).wait()

      def repack(s):
        gb = gbufs[s]
        for j2 in range(_NB):
          for l in range(_L):
            for h in range(_E // 16):
              obuf[j2, l, pl.ds(h * 16, 16)] = gb[j2 * _GROW + l,
                                                  pl.ds(h * 16, 16)]

      def store(k):
        pltpu.sync_copy(obuf, out_hbm.at[pl.ds(b0 + k * _NB, _NB)])

      gather(0, 0)
      gather(1, 1)

      def half(i, k, s):
        wait_gather(s)
        repack(s)
        store(k)
        pl.when(i < _NBLK // 2 - 1)(lambda: gather(k + 2, s))

      def body(i, _):
        half(i, 2 * i, 0)
        half(i, 2 * i + 1, 1)
        return 0

      lax.fori_loop(0, _NBLK // 2, body, 0)

    run_table(src_idx, src_table, src_out)
    run_table(tar_idx, tar_table, tar_out)

  return emb_kernel


_EMB = _make_kernel()


@jax.jit
def kernel(src_idx, tar_idx, src_table, tar_table):
  sp = jnp.pad(src_table, ((0, 0), (0, 128 - _E)))
  tp = jnp.pad(tar_table, ((0, 0), (0, 128 - _E)))
  return _EMB(src_idx, tar_idx, sp, tp)


# speedup vs baseline: 1.0681x; 1.0222x over previous
import functools

import jax
import jax.numpy as jnp
from jax import lax
from jax.experimental import pallas as pl
from jax.experimental.pallas import tpu as pltpu
from jax.experimental.pallas import tpu_sc as plsc

_B, _L, _E = 4096, 50, 64
_V = 100000
_NC, _NS = 2, 16
_NW = _NC * _NS
_BPW = _B // _NW
_NB = 2
_NBLK = _BPW // _NB
_GROW = 56


def _make_kernel():
  mesh = plsc.VectorSubcoreMesh(
      core_axis_name="c", subcore_axis_name="s",
      num_cores=_NC, num_subcores=_NS)

  @functools.partial(
      pl.kernel,
      out_type=(
          jax.ShapeDtypeStruct((_B, _L, _E), jnp.float32),
          jax.ShapeDtypeStruct((_B, _L, _E), jnp.float32),
      ),
      mesh=mesh,
      compiler_params=pltpu.CompilerParams(use_tc_tiling_on_sc=True),
      scratch_types=[
          pltpu.VMEM((_BPW, _L), jnp.int32),
          pltpu.VMEM((_NB * _GROW, 128), jnp.float32),
          pltpu.VMEM((_NB * _GROW, 128), jnp.float32),
          pltpu.VMEM((_NB, _L, _E), jnp.float32),
          pltpu.VMEM((_NB, _L, _E), jnp.float32),
          pltpu.SemaphoreType.DMA,
          pltpu.SemaphoreType.DMA,
          pltpu.SemaphoreType.DMA,
          pltpu.SemaphoreType.DMA,
      ],
  )
  def emb_kernel(src_idx, tar_idx, src_table, tar_table,
                 src_out, tar_out, idx_all, gbuf0, gbuf1, obuf0, obuf1,
                 g0, g1, o0, o1):
    wid = lax.axis_index("s") * _NC + lax.axis_index("c")
    b0 = wid * _BPW
    gbufs = (gbuf0, gbuf1)
    obufs = (obuf0, obuf1)
    gsems = (g0, g1)
    osems = (o0, o1)

    def run_table(idx_hbm, table_hbm, out_hbm):
      pltpu.sync_copy(idx_hbm.at[pl.ds(b0, _BPW), :], idx_all)

      def gather(k, s):
        for j2 in range(_NB):
          pltpu.async_copy(
              table_hbm.at[idx_all.at[k * _NB + j2, pl.ds(0, _L)]],
              gbufs[s].at[pl.ds(j2 * _GROW, _L)], gsems[s])

      def wait_gather(s):
        for j2 in range(_NB):
          pltpu.make_async_copy(
              table_hbm.at[idx_all.at[j2, pl.ds(0, _L)]],
              gbufs[s].at[pl.ds(j2 * _GROW, _L)], gsems[s]).wait()

      def repack(s):
        gb, ob = gbufs[s], obufs[s]
        for j2 in range(_NB):
          for l in range(_L):
            for h in range(_E // 16):
              ob[j2, l, pl.ds(h * 16, 16)] = gb[j2 * _GROW + l,
                                                pl.ds(h * 16, 16)]

      def store(k, s):
        pltpu.async_copy(obufs[s], out_hbm.at[pl.ds(b0 + k * _NB, _NB)],
                         osems[s])

      def wait_store(s):
        pltpu.make_async_copy(obufs[s], out_hbm.at[pl.ds(b0, _NB)],
                              osems[s]).wait()

      gather(0, 0)
      gather(1, 1)

      def half(i, k, s):
        wait_gather(s)
        pl.when(i > 0)(lambda: wait_store(s))
        repack(s)
        store(k, s)
        pl.when(i < _NBLK // 2 - 1)(lambda: gather(k + 2, s))

      def body(i, _):
        half(i, 2 * i, 0)
        half(i, 2 * i + 1, 1)
        return 0

      lax.fori_loop(0, _NBLK // 2, body, 0)

      wait_store(0)
      wait_store(1)

    run_table(src_idx, src_table, src_out)
    run_table(tar_idx, tar_table, tar_out)

  return emb_kernel


_EMB = _make_kernel()


@jax.jit
def kernel(src_idx, tar_idx, src_table, tar_table):
  sp = jnp.pad(src_table, ((0, 0), (0, 128 - _E)))
  tp = jnp.pad(tar_table, ((0, 0), (0, 128 - _E)))
  return _EMB(src_idx, tar_idx, sp, tp)


# speedup vs baseline: 1.0698x; 1.0016x over previous
import functools

import jax
import jax.numpy as jnp
from jax import lax
from jax.experimental import pallas as pl
from jax.experimental.pallas import tpu as pltpu
from jax.experimental.pallas import tpu_sc as plsc

_B, _L, _E = 4096, 50, 64
_N = _B * _L
_V = 100000
_NC, _NS = 2, 16
_NW = _NC * _NS
_BPW = _B // _NW
_NB = 4
_NBLK = _BPW // _NB
_ROWS = _NB * _L


def _make_kernel():
  mesh = plsc.VectorSubcoreMesh(
      core_axis_name="c", subcore_axis_name="s",
      num_cores=_NC, num_subcores=_NS)

  @functools.partial(
      pl.kernel,
      out_type=(
          jax.ShapeDtypeStruct((_B, _L, _E), jnp.float32),
          jax.ShapeDtypeStruct((_B, _L, _E), jnp.float32),
      ),
      mesh=mesh,
      compiler_params=pltpu.CompilerParams(use_tc_tiling_on_sc=True),
      scratch_types=[
          pltpu.VMEM((_BPW * _L,), jnp.int32),
          pltpu.VMEM((_ROWS, 128), jnp.float32),
          pltpu.VMEM((_ROWS, 128), jnp.float32),
          pltpu.VMEM((_NB, _L, _E), jnp.float32),
          pltpu.VMEM((_NB, _L, _E), jnp.float32),
          pltpu.SemaphoreType.DMA,
          pltpu.SemaphoreType.DMA,
          pltpu.SemaphoreType.DMA,
          pltpu.SemaphoreType.DMA,
      ],
  )
  def emb_kernel(src_idx, tar_idx, src_table, tar_table,
                 src_out, tar_out, idx_all,
                 gbuf0, gbuf1, obuf0, obuf1, g0, g1, o0, o1):
    wid = lax.axis_index("s") * _NC + lax.axis_index("c")
    b0 = wid * _BPW
    gbufs = (gbuf0, gbuf1)
    obufs = (obuf0, obuf1)
    gsems = (g0, g1)
    osems = (o0, o1)

    def run_table(idx_hbm, table_hbm, out_hbm):
      pltpu.sync_copy(idx_hbm.at[pl.ds(b0 * _L, _BPW * _L)], idx_all)

      def gather(k, s):
        pltpu.async_copy(
            table_hbm.at[idx_all.at[pl.ds(k * _ROWS, _ROWS)]],
            gbufs[s], gsems[s])

      def wait_gather(s):
        pltpu.make_async_copy(
            table_hbm.at[pl.ds(0, _ROWS)], gbufs[s], gsems[s]).wait()

      def repack(s):
        gb, ob = gbufs[s], obufs[s]
        for j in range(_ROWS):
          for h in range(_E // 16):
            ob[j // _L, j % _L, pl.ds(h * 16, 16)] = gb[j, pl.ds(h * 16, 16)]

      def store(k, s):
        pltpu.async_copy(obufs[s], out_hbm.at[pl.ds(b0 + k * _NB, _NB)],
                         osems[s])

      def wait_store(s):
        pltpu.make_async_copy(obufs[s], out_hbm.at[pl.ds(b0, _NB)],
                              osems[s]).wait()

      gather(0, 0)
      gather(1, 1)

      def half(i, k, s):
        wait_gather(s)
        pl.when(i > 0)(lambda: wait_store(s))
        repack(s)
        store(k, s)
        pl.when(i < _NBLK // 2 - 1)(lambda: gather(k + 2, s))

      def body(i, _):
        half(i, 2 * i, 0)
        half(i, 2 * i + 1, 1)
        return 0

      lax.fori_loop(0, _NBLK // 2, body, 0)

      wait_store(0)
      wait_store(1)

    run_table(src_idx, src_table, src_out)
    run_table(tar_idx, tar_table, tar_out)

  return emb_kernel


_EMB = _make_kernel()


@jax.jit
def kernel(src_idx, tar_idx, src_table, tar_table):
  sp = jnp.pad(src_table, ((0, 0), (0, 128 - _E)))
  tp = jnp.pad(tar_table, ((0, 0), (0, 128 - _E)))
  return _EMB(src_idx.reshape(_N), tar_idx.reshape(_N), sp, tp)
